# baseline (speedup 1.0000x reference)
import functools

import jax
import jax.numpy as jnp
from jax import lax
from jax.experimental import pallas as pl
from jax.experimental.pallas import tpu as pltpu
from jax.experimental.pallas import tpu_sc as plsc

NC = 2
NS = 16
NW = NC * NS
L = 16

D = 64
CH = 64
NR = 6 * D


def _sc_score(B, n_chunks):
    mesh = plsc.VectorSubcoreMesh(
        core_axis_name="c", subcore_axis_name="s", num_cores=NC, num_subcores=NS
    )
    q_per_w = B // NW
    assert q_per_w == n_chunks * CH and n_chunks % 2 == 0

    @functools.partial(
        pl.kernel,
        out_type=jax.ShapeDtypeStruct((B,), jnp.float32),
        mesh=mesh,
        compiler_params=pltpu.CompilerParams(needs_layout_passes=False),
        scratch_types=dict(
            soidx=pltpu.VMEM((2 * q_per_w,), jnp.int32),
            rtidx=pltpu.VMEM((2 * q_per_w,), jnp.int32),
            sobuf=[pltpu.VMEM((2 * CH, 2 * D), jnp.float32) for _ in range(2)],
            rtbuf=[pltpu.VMEM((2 * CH, 256), jnp.int32) for _ in range(2)],
            part=pltpu.VMEM((L, L), jnp.float32),
            res=pltpu.VMEM((CH,), jnp.float32),
            sem=[pltpu.SemaphoreType.DMA for _ in range(2)],
        ),
    )
    def score(idx_h, Ecat, RTcat, out_h, *, soidx, rtidx, sobuf, rtbuf,
              part, res, sem):
        wid = lax.axis_index("s") * NC + lax.axis_index("c")
        base = wid * q_per_w
        pltpu.sync_copy(idx_h.at[pl.ds(2 * base, 2 * q_per_w)], soidx)
        pltpu.sync_copy(idx_h.at[pl.ds(2 * B + 2 * base, 2 * q_per_w)], rtidx)

        def fetch(ch, slot):
            pltpu.async_copy(
                Ecat.at[soidx.at[pl.ds(2 * CH * ch, 2 * CH)]], sobuf[slot],
                sem[slot])
            pltpu.async_copy(
                RTcat.at[rtidx.at[pl.ds(2 * CH * ch, 2 * CH)]], rtbuf[slot],
                sem[slot])

        def drain(slot):
            pltpu.make_async_copy(
                Ecat.at[soidx.at[pl.ds(0, 2 * CH)]], sobuf[slot],
                sem[slot]).wait()
            pltpu.make_async_copy(
                RTcat.at[rtidx.at[pl.ds(0, 2 * CH)]], rtbuf[slot],
                sem[slot]).wait()

        def compute(slot):
            so, rt = sobuf[slot], rtbuf[slot]

            def unpk(ref_row, col):
                words = rt[ref_row, pl.ds(col // 2, L)]
                return plsc.unpack(
                    plsc.bitcast(words, jnp.bfloat16),
                    format=plsc.PackFormat.INTERLEAVED,
                    preferred_element_type=jnp.float32)

            def gbody(qg, carry):
                for j in range(L):
                    q = qg * L + j
                    acc = jnp.zeros((L,), jnp.float32)
                    for gp in range(2):
                        o0 = 2 * L * gp
                        sr = (so[q, pl.ds(o0, L)], so[q, pl.ds(o0 + L, L)])
                        si = (so[q, pl.ds(D + o0, L)],
                              so[q, pl.ds(D + o0 + L, L)])
                        orv = (so[CH + q, pl.ds(o0, L)],
                               so[CH + q, pl.ds(o0 + L, L)])
                        oi = (so[CH + q, pl.ds(D + o0, L)],
                              so[CH + q, pl.ds(D + o0 + L, L)])
                        rr = unpk(q, o0)
                        ri = unpk(q, D + o0)
                        rsr = unpk(q, 2 * D + o0)
                        rsi = unpk(q, 3 * D + o0)
                        ror = unpk(q, 4 * D + o0)
                        roi = unpk(q, 5 * D + o0)
                        tr = unpk(CH + q, o0)
                        ti = unpk(CH + q, D + o0)
                        for h in range(2):
                            a = (sr[h] * rr[h] - si[h] * ri[h]
                                 + tr[h] * ror[h] - ti[h] * roi[h])
                            b = (sr[h] * ri[h] + si[h] * rr[h]
                                 + tr[h] * roi[h] + ti[h] * ror[h])
                            c = sr[h] * rsr[h] - si[h] * rsi[h]
                            d = sr[h] * rsi[h] + si[h] * rsr[h]
                            acc = acc + (a * orv[h] + b * oi[h]
                                         + c * tr[h] + d * ti[h])
                    part[j] = plsc.cumsum(acc)
                rows = lax.iota(jnp.int32, L)
                cols = jnp.full((L,), L - 1, jnp.int32)
                res[pl.ds(qg * L, L)] = plsc.load_gather(part, [rows, cols])
                return carry

            lax.fori_loop(0, CH // L, gbody, 0)

        npairs = n_chunks // 2
        fetch(0, 0)

        def pair_body(p, carry):
            c0 = 2 * p
            drain(0)
            fetch(c0 + 1, 1)
            compute(0)
            pltpu.sync_copy(res, out_h.at[pl.ds(base + c0 * CH, CH)])
            drain(1)

            @pl.when(p + 1 < npairs)
            def _():
                fetch(c0 + 2, 0)

            compute(1)
            pltpu.sync_copy(res, out_h.at[pl.ds(base + (c0 + 1) * CH, CH)])
            return carry

        lax.fori_loop(0, npairs, pair_body, 0)

    return score


def _fuse_entity_tables(e_re_t, e_im_t):
    V = e_re_t.shape[1]
    C = 12288

    def body(re_ref, im_ref, out_ref):
        eye = (
            lax.broadcasted_iota(jnp.int32, (D, D), 0)
            == lax.broadcasted_iota(jnp.int32, (D, D), 1)
        ).astype(jnp.float32)
        dims = (((0,), (0,)), ((), ()))
        out_ref[:, 0:D] = lax.dot_general(
            re_ref[...], eye, dims,
            preferred_element_type=jnp.float32, precision=lax.Precision.DEFAULT)
        out_ref[:, D:2 * D] = lax.dot_general(
            im_ref[...], eye, dims,
            preferred_element_type=jnp.float32, precision=lax.Precision.DEFAULT)

    return pl.pallas_call(
        body,
        grid=(pl.cdiv(V, C),),
        in_specs=[
            pl.BlockSpec((D, C), lambda i: (0, i)),
            pl.BlockSpec((D, C), lambda i: (0, i)),
        ],
        out_specs=pl.BlockSpec((C, 2 * D), lambda i: (i, 0)),
        out_shape=jax.ShapeDtypeStruct((V, 2 * D), jnp.float32),
    )(e_re_t, e_im_t)


def _fuse_rt_tables(r_views, t_views):
    n_rel = r_views[0].shape[1]
    n_t = t_views[0].shape[1]

    def to_rows(ref):
        eye = (
            lax.broadcasted_iota(jnp.int32, (D, D), 0)
            == lax.broadcasted_iota(jnp.int32, (D, D), 1)
        ).astype(jnp.float32)
        x = lax.dot_general(
            ref[...], eye, (((0,), (0,)), ((), ())),
            preferred_element_type=jnp.float32, precision=lax.Precision.DEFAULT)
        bits = lax.bitcast_convert_type(x, jnp.uint32)
        rnd = ((bits + 0x8000) >> 16) & 0xFFFF
        words = []
        for k in range(2):
            lo = rnd[:, 32 * k:32 * k + 16]
            hi = rnd[:, 32 * k + 16:32 * k + 32]
            words.append(((hi << 16) | lo).astype(jnp.int32))
        return jnp.concatenate(words, axis=1)

    def body(*refs):
        out_ref = refs[-1]
        out_ref[...] = jnp.zeros(out_ref.shape, jnp.int32)
        for st in range(6):
            out_ref[0:n_rel, 32 * st:32 * (st + 1)] = to_rows(refs[st])
        for st in range(2):
            out_ref[n_rel:n_rel + n_t, 32 * st:32 * (st + 1)] = to_rows(
                refs[6 + st])

    return pl.pallas_call(
        body,
        in_specs=[pl.BlockSpec(v.shape, lambda: (0, 0))
                  for v in (*r_views, *t_views)],
        out_specs=pl.BlockSpec((n_rel + n_t, 256), lambda: (0, 0)),
        out_shape=jax.ShapeDtypeStruct((n_rel + n_t, 256), jnp.int32),
    )(*r_views, *t_views)


def kernel(s, r, o, t, E_im, E_re, R_im, R_re, Rs_im, Rs_re, Ro_im, Ro_re,
           Ts_im, Ts_re, To_im, To_re):
    del To_im, To_re
    B = s.shape[0]
    n_rel = R_im.shape[0]
    so_idx = jnp.stack(
        [s.reshape(-1, CH), o.reshape(-1, CH)], axis=1
    ).reshape(2 * B)
    rt_idx = jnp.stack(
        [r.reshape(-1, CH), t[:, 0, 0].reshape(-1, CH) + n_rel], axis=1
    ).reshape(2 * B)
    idx_h = jnp.concatenate([so_idx, rt_idx]).astype(jnp.int32)

    Ecat = _fuse_entity_tables(E_re.T, E_im.T)
    RTcat = _fuse_rt_tables(
        (R_re.T, R_im.T, Rs_re.T, Rs_im.T, Ro_re.T, Ro_im.T),
        (Ts_re.T, Ts_im.T))

    score = _sc_score(B, B // (NW * CH))
    out = score(idx_h, Ecat, RTcat)
    return out.reshape(B, 1)


# speedup vs baseline: 1.0042x; 1.0042x over previous
import functools

import jax
import jax.numpy as jnp
from jax import lax
from jax.experimental import pallas as pl
from jax.experimental.pallas import tpu as pltpu
from jax.experimental.pallas import tpu_sc as plsc

NC = 2
NS = 16
NW = NC * NS
L = 16

D = 64
CH = 64
NR = 6 * D


def _sc_score(B, n_chunks):
    mesh = plsc.VectorSubcoreMesh(
        core_axis_name="c", subcore_axis_name="s", num_cores=NC, num_subcores=NS
    )
    q_per_w = B // NW
    assert q_per_w == n_chunks * CH and n_chunks % 2 == 0

    @functools.partial(
        pl.kernel,
        out_type=jax.ShapeDtypeStruct((B,), jnp.float32),
        mesh=mesh,
        compiler_params=pltpu.CompilerParams(needs_layout_passes=False),
        scratch_types=dict(
            soidx=pltpu.VMEM((2 * q_per_w,), jnp.int32),
            rtidx=pltpu.VMEM((2 * q_per_w,), jnp.int32),
            sobuf=[pltpu.VMEM((2 * CH, 2 * D), jnp.float32) for _ in range(2)],
            rtbuf=[pltpu.VMEM((2 * CH, 256), jnp.int32) for _ in range(2)],
            part=pltpu.VMEM((L, L), jnp.float32),
            res=pltpu.VMEM((CH,), jnp.float32),
            sem=[pltpu.SemaphoreType.DMA for _ in range(2)],
        ),
    )
    def score(idx_h, Ecat, RTcat, out_h, *, soidx, rtidx, sobuf, rtbuf,
              part, res, sem):
        wid = lax.axis_index("s") * NC + lax.axis_index("c")
        base = wid * q_per_w
        pltpu.sync_copy(idx_h.at[pl.ds(2 * base, 2 * q_per_w)], soidx)
        pltpu.sync_copy(idx_h.at[pl.ds(2 * B + 2 * base, 2 * q_per_w)], rtidx)

        def fetch(ch, slot):
            pltpu.async_copy(
                Ecat.at[soidx.at[pl.ds(2 * CH * ch, 2 * CH)]], sobuf[slot],
                sem[slot])
            pltpu.async_copy(
                RTcat.at[rtidx.at[pl.ds(2 * CH * ch, 2 * CH)]], rtbuf[slot],
                sem[slot])

        def drain(slot):
            pltpu.make_async_copy(
                Ecat.at[soidx.at[pl.ds(0, 2 * CH)]], sobuf[slot],
                sem[slot]).wait()
            pltpu.make_async_copy(
                RTcat.at[rtidx.at[pl.ds(0, 2 * CH)]], rtbuf[slot],
                sem[slot]).wait()

        def compute(slot):
            so, rt = sobuf[slot], rtbuf[slot]

            def unpk(ref_row, col):
                words = rt[ref_row, pl.ds(col // 2, L)]
                return plsc.unpack(
                    plsc.bitcast(words, jnp.bfloat16),
                    format=plsc.PackFormat.INTERLEAVED,
                    preferred_element_type=jnp.float32)

            def gbody(qg, carry):
                for j in range(L):
                    q = qg * L + j
                    acc = jnp.zeros((L,), jnp.float32)
                    for gp in range(2):
                        o0 = 2 * L * gp
                        sr = (so[q, pl.ds(o0, L)], so[q, pl.ds(o0 + L, L)])
                        si = (so[q, pl.ds(D + o0, L)],
                              so[q, pl.ds(D + o0 + L, L)])
                        orv = (so[CH + q, pl.ds(o0, L)],
                               so[CH + q, pl.ds(o0 + L, L)])
                        oi = (so[CH + q, pl.ds(D + o0, L)],
                              so[CH + q, pl.ds(D + o0 + L, L)])
                        rr = unpk(q, o0)
                        ri = unpk(q, D + o0)
                        rsr = unpk(q, 2 * D + o0)
                        rsi = unpk(q, 3 * D + o0)
                        ror = unpk(q, 4 * D + o0)
                        roi = unpk(q, 5 * D + o0)
                        tr = unpk(CH + q, o0)
                        ti = unpk(CH + q, D + o0)
                        for h in range(2):
                            a = (sr[h] * rr[h] - si[h] * ri[h]
                                 + tr[h] * ror[h] - ti[h] * roi[h])
                            b = (sr[h] * ri[h] + si[h] * rr[h]
                                 + tr[h] * roi[h] + ti[h] * ror[h])
                            c = sr[h] * rsr[h] - si[h] * rsi[h]
                            d = sr[h] * rsi[h] + si[h] * rsr[h]
                            acc = acc + (a * orv[h] + b * oi[h]
                                         + c * tr[h] + d * ti[h])
                    part[j] = plsc.cumsum(acc)
                rows = lax.iota(jnp.int32, L)
                cols = jnp.full((L,), L - 1, jnp.int32)
                res[pl.ds(qg * L, L)] = plsc.load_gather(part, [rows, cols])
                return carry

            lax.fori_loop(0, CH // L, gbody, 0)

        npairs = n_chunks // 2
        fetch(0, 0)

        def pair_body(p, carry):
            c0 = 2 * p
            drain(0)
            fetch(c0 + 1, 1)
            compute(0)
            pltpu.sync_copy(res, out_h.at[pl.ds(base + c0 * CH, CH)])
            drain(1)

            @pl.when(p + 1 < npairs)
            def _():
                fetch(c0 + 2, 0)

            compute(1)
            pltpu.sync_copy(res, out_h.at[pl.ds(base + (c0 + 1) * CH, CH)])
            return carry

        lax.fori_loop(0, npairs, pair_body, 0)

    return score


def _fuse_entity_tables(e_re_t, e_im_t):
    V = e_re_t.shape[1]
    C = 8192

    def body(re_ref, im_ref, out_ref):
        eye = (
            lax.broadcasted_iota(jnp.int32, (D, D), 0)
            == lax.broadcasted_iota(jnp.int32, (D, D), 1)
        ).astype(jnp.float32)
        dims = (((0,), (0,)), ((), ()))
        out_ref[:, 0:D] = lax.dot_general(
            re_ref[...], eye, dims,
            preferred_element_type=jnp.float32, precision=lax.Precision.DEFAULT)
        out_ref[:, D:2 * D] = lax.dot_general(
            im_ref[...], eye, dims,
            preferred_element_type=jnp.float32, precision=lax.Precision.DEFAULT)

    return pl.pallas_call(
        body,
        grid=(pl.cdiv(V, C),),
        in_specs=[
            pl.BlockSpec((D, C), lambda i: (0, i)),
            pl.BlockSpec((D, C), lambda i: (0, i)),
        ],
        out_specs=pl.BlockSpec((C, 2 * D), lambda i: (i, 0)),
        out_shape=jax.ShapeDtypeStruct((V, 2 * D), jnp.float32),
    )(e_re_t, e_im_t)


def _fuse_rt_tables(r_views, t_views):
    n_rel = r_views[0].shape[1]
    n_t = t_views[0].shape[1]

    def to_rows(ref):
        eye = (
            lax.broadcasted_iota(jnp.int32, (D, D), 0)
            == lax.broadcasted_iota(jnp.int32, (D, D), 1)
        ).astype(jnp.float32)
        x = lax.dot_general(
            ref[...], eye, (((0,), (0,)), ((), ())),
            preferred_element_type=jnp.float32, precision=lax.Precision.DEFAULT)
        bits = lax.bitcast_convert_type(x, jnp.uint32)
        rnd = ((bits + 0x8000) >> 16) & 0xFFFF
        words = []
        for k in range(2):
            lo = rnd[:, 32 * k:32 * k + 16]
            hi = rnd[:, 32 * k + 16:32 * k + 32]
            words.append(((hi << 16) | lo).astype(jnp.int32))
        return jnp.concatenate(words, axis=1)

    def body(*refs):
        out_ref = refs[-1]
        out_ref[...] = jnp.zeros(out_ref.shape, jnp.int32)
        for st in range(6):
            out_ref[0:n_rel, 32 * st:32 * (st + 1)] = to_rows(refs[st])
        for st in range(2):
            out_ref[n_rel:n_rel + n_t, 32 * st:32 * (st + 1)] = to_rows(
                refs[6 + st])

    return pl.pallas_call(
        body,
        in_specs=[pl.BlockSpec(v.shape, lambda: (0, 0))
                  for v in (*r_views, *t_views)],
        out_specs=pl.BlockSpec((n_rel + n_t, 256), lambda: (0, 0)),
        out_shape=jax.ShapeDtypeStruct((n_rel + n_t, 256), jnp.int32),
    )(*r_views, *t_views)


def kernel(s, r, o, t, E_im, E_re, R_im, R_re, Rs_im, Rs_re, Ro_im, Ro_re,
           Ts_im, Ts_re, To_im, To_re):
    del To_im, To_re
    B = s.shape[0]
    n_rel = R_im.shape[0]
    so_idx = jnp.stack(
        [s.reshape(-1, CH), o.reshape(-1, CH)], axis=1
    ).reshape(2 * B)
    rt_idx = jnp.stack(
        [r.reshape(-1, CH), t[:, 0, 0].reshape(-1, CH) + n_rel], axis=1
    ).reshape(2 * B)
    idx_h = jnp.concatenate([so_idx, rt_idx]).astype(jnp.int32)

    Ecat = _fuse_entity_tables(E_re.T, E_im.T)
    RTcat = _fuse_rt_tables(
        (R_re.T, R_im.T, Rs_re.T, Rs_im.T, Ro_re.T, Ro_im.T),
        (Ts_re.T, Ts_im.T))

    score = _sc_score(B, B // (NW * CH))
    out = score(idx_h, Ecat, RTcat)
    return out.reshape(B, 1)
